# Initial kernel scaffold; baseline (speedup 1.0000x reference)
import jax
import jax.numpy as jnp
from jax.experimental import pallas as pl


def kernel(h, pairs_0, pairs_1, degrees_0, degrees_1, scatter_idx, W_lin, b_lin, W_t, b_t, eps):
    raise NotImplementedError("write your pallas kernel here")


# speedup vs baseline: 2.6073x; 2.6073x over previous
import functools

import jax
import jax.numpy as jnp
from jax import lax
from jax.experimental import pallas as pl
from jax.experimental.pallas import tpu as pltpu
from jax.experimental.pallas import tpu_sc as plsc

LANES = 16
BLK = 128
SUP = 8
NC, NS = 2, 16
NW = NC * NS


def _mm_body(h_ref, wl_ref, bl_ref, wh_ref, bth_ref, h3_ref, p_ref):
    hb = h_ref[...]
    h3_ref[...] = jnp.dot(hb, wl_ref[...],
                          preferred_element_type=jnp.float32) + bl_ref[...]
    p_ref[...] = jnp.dot(hb, wh_ref[...],
                         preferred_element_type=jnp.float32) + bth_ref[...]


def _combine_body(n, h3_ref, parts_ref, eps_ref, out_ref):
    scale = 1.0 + eps_ref[0]
    sl = pl.ds(0, n)
    out_ref[...] = h3_ref[...] + scale * (parts_ref[0, sl] + parts_ref[1, sl])


def _make_sc_kernel(n_nodes, d, n_blocks_per_worker, n_acc_rows, zrows):
    mesh = plsc.VectorSubcoreMesh(core_axis_name="c", subcore_axis_name="s")
    nsup = n_blocks_per_worker // SUP

    @functools.partial(
        pl.kernel,
        out_type=jax.ShapeDtypeStruct((NC, n_acc_rows, d), jnp.float32),
        mesh=mesh,
        scratch_types=[
            pltpu.VMEM((SUP, BLK), jnp.int32),
            pltpu.VMEM((SUP, BLK), jnp.int32),
            pltpu.VMEM((SUP, BLK), jnp.int32),
            pltpu.VMEM((3, SUP * BLK), jnp.float32),
            pltpu.VMEM((3, SUP * BLK), jnp.float32),
            pltpu.VMEM((BLK, d), jnp.float32),
            pltpu.VMEM((BLK, d), jnp.float32),
            pltpu.VMEM((3, d), jnp.float32),
            pltpu.VMEM_SHARED((n_acc_rows, d), jnp.float32),
            pltpu.SemaphoreType.DMA,
            pltpu.SemaphoreType.DMA,
        ],
    )
    def sc_kernel(p_hbm, p0_hbm, p1_hbm, si_hbm, d0_hbm, d1_hbm, wd_hbm,
                  z_hbm, out_hbm, idx0_v, idx1_v, sidx_v, deg0_v, deg1_v,
                  rows0_v, rows1_v, wd_v, acc, sem0, sem1):
        c = lax.axis_index("c")
        s = lax.axis_index("s")
        wid = c * NS + s

        pltpu.sync_copy(z_hbm, acc.at[pl.ds(s * zrows, zrows)])
        pltpu.sync_copy(wd_hbm, wd_v)
        plsc.subcore_barrier()

        wch = [[wd_v[k, pl.ds(ch * LANES, LANES)] for ch in range(d // LANES)]
               for k in range(3)]

        def superblock(sb, carry):
            row0 = (wid * n_blocks_per_worker) + sb * SUP
            sb_global = wid * nsup + sb
            pltpu.sync_copy(p0_hbm.at[pl.ds(row0, SUP)], idx0_v)
            pltpu.sync_copy(p1_hbm.at[pl.ds(row0, SUP)], idx1_v)
            pltpu.sync_copy(si_hbm.at[pl.ds(row0, SUP)], sidx_v)
            pltpu.sync_copy(d0_hbm.at[sb_global], deg0_v)
            pltpu.sync_copy(d1_hbm.at[sb_global], deg1_v)

            def block(j, carry2):
                cp0 = pltpu.async_copy(p_hbm.at[idx0_v.at[j]], rows0_v, sem0)
                cp1 = pltpu.async_copy(p_hbm.at[idx1_v.at[j]], rows1_v, sem1)
                cp0.wait()
                cp1.wait()

                def group(g, carry3):
                    goff = j * BLK + g * LANES
                    dsum = [deg0_v[k, pl.ds(goff, LANES)]
                            + deg1_v[k, pl.ds(goff, LANES)] for k in range(3)]
                    for el in range(LANES):
                        e = g * LANES + el
                        ds0, ds1, ds2 = dsum[0][el], dsum[1][el], dsum[2][el]
                        for ch in range(d // LANES):
                            sl = pl.ds(ch * LANES, LANES)
                            v = rows0_v[e, sl] + rows1_v[e, sl]
                            v = v + ds0 * wch[0][ch]
                            v = v + ds1 * wch[1][ch]
                            v = v + ds2 * wch[2][ch]
                            rows0_v[e, sl] = jnp.maximum(v, 0.0)
                    return carry3

                lax.fori_loop(0, BLK // LANES, group, 0, unroll=False)
                pltpu.sync_copy(rows0_v, acc.at[sidx_v.at[j]], add=True)
                return carry2

            lax.fori_loop(0, SUP, block, 0, unroll=False)
            return carry

        lax.fori_loop(0, nsup, superblock, 0, unroll=False)

        plsc.subcore_barrier()
        pltpu.sync_copy(acc.at[pl.ds(s * zrows, zrows)],
                        out_hbm.at[c, pl.ds(s * zrows, zrows)])

    return sc_kernel


def kernel(h, pairs_0, pairs_1, degrees_0, degrees_1, scatter_idx,
           W_lin, b_lin, W_t, b_t, eps):
    n, d_in = h.shape
    d_out = W_lin.shape[1]
    e = pairs_0.shape[0]

    w_h = W_t[:d_in]
    w_d = W_t[d_in:]
    h3, p_tab = pl.pallas_call(
        _mm_body,
        out_shape=(jax.ShapeDtypeStruct((n, d_out), jnp.float32),
                   jax.ShapeDtypeStruct((n, d_out), jnp.float32)),
    )(h, W_lin, b_lin.reshape(1, d_out), w_h, (0.5 * b_t).reshape(1, d_out))

    chunk = NW * SUP * BLK
    e_pad = -(-e // chunk) * chunk
    pad = e_pad - e
    zrows = -(-(n + 1) // (NS * 8)) * 8
    n_dump = NS * zrows
    p0 = jnp.pad(pairs_0, (0, pad)).reshape(e_pad // BLK, BLK)
    p1 = jnp.pad(pairs_1, (0, pad)).reshape(e_pad // BLK, BLK)
    si = jnp.pad(scatter_idx, (0, pad), constant_values=n).reshape(
        e_pad // BLK, BLK)
    nsb = e_pad // (SUP * BLK)
    d0 = jnp.pad(degrees_0, ((0, pad), (0, 0))).T.reshape(
        3, nsb, SUP * BLK).transpose(1, 0, 2)
    d1 = jnp.pad(degrees_1, ((0, pad), (0, 0))).T.reshape(
        3, nsb, SUP * BLK).transpose(1, 0, 2)
    zeros = jnp.zeros((zrows, d_out), jnp.float32)

    sc = _make_sc_kernel(n, d_out, e_pad // BLK // NW, n_dump, zrows)
    partials = sc(p_tab, p0, p1, si, d0, d1, w_d, zeros)

    out = pl.pallas_call(
        functools.partial(_combine_body, n),
        in_specs=[pl.BlockSpec(memory_space=pltpu.VMEM),
                  pl.BlockSpec(memory_space=pltpu.VMEM),
                  pl.BlockSpec(memory_space=pltpu.SMEM)],
        out_shape=jax.ShapeDtypeStruct((n, d_out), jnp.float32),
    )(h3, partials, eps)
    return out
